# baseline (speedup 1.0000x reference)
import functools

import jax
import jax.numpy as jnp
from jax import lax
from jax.experimental import pallas as pl
from jax.experimental.pallas import tpu as pltpu
from jax.experimental.pallas import tpu_sc as plsc

EMBED = 64
LANES = 16
CHUNK = 128
SEQ = 200
BATCH = 1024
TOTAL = SEQ * BATCH
NCHUNKS = TOTAL // CHUNK
ALPHA_C = 0.2
BETA_C = 0.8


def _build_sc_kernel():
    info = plsc.get_sparse_core_info()
    nc, ns = info.num_cores, info.num_subcores
    nw = nc * ns
    per_w = NCHUNKS // nw

    mesh = plsc.VectorSubcoreMesh(core_axis_name="c", subcore_axis_name="s")

    @functools.partial(
        pl.kernel,
        mesh=mesh,
        compiler_params=pltpu.CompilerParams(needs_layout_passes=False),
        out_type=jax.ShapeDtypeStruct((SEQ, EMBED, BATCH), jnp.float32),
        scratch_types=[
            pltpu.VMEM((CHUNK,), jnp.int32),
            pltpu.VMEM((CHUNK,), jnp.int32),
            pltpu.VMEM((CHUNK,), jnp.int32),
            pltpu.VMEM((CHUNK, 2 * EMBED), jnp.float32),
            pltpu.VMEM((EMBED, CHUNK), jnp.float32),
            pltpu.VMEM((EMBED,), jnp.float32),
            pltpu.SemaphoreType.DMA,
        ],
    )
    def sc_kernel(ids_hbm, tab2_hbm, pos_hbm, out_hbm,
                  idx_v, idx2_v, base_v, rows_v, out_v, pos_v, sem):
        wid = lax.axis_index("s") * nc + lax.axis_index("c")

        def chunk_body(i, carry):
            c = wid * per_w + i
            s_pos = c >> 3
            b0 = (c & 7) * CHUNK
            pltpu.sync_copy(ids_hbm.at[c], idx_v)
            pltpu.sync_copy(pos_hbm.at[s_pos], pos_v)
            for k in range(CHUNK // LANES):
                v = idx_v[pl.ds(k * LANES, LANES)]
                idx2_v[pl.ds(k * LANES, LANES)] = v >> 1
                base_v[pl.ds(k * LANES, LANES)] = (v & 1) * EMBED
            pltpu.async_copy(tab2_hbm.at[idx2_v], rows_v, sem).wait()

            def d_body(d, dcarry):
                pd = plsc.load_gather(
                    pos_v, [jnp.full((LANES,), d, jnp.int32)]) * ALPHA_C
                for g in range(CHUNK // LANES):
                    rr = lax.iota(jnp.int32, LANES) + (g * LANES)
                    cc = base_v[pl.ds(g * LANES, LANES)] + d
                    vals = plsc.load_gather(rows_v, [rr, cc])
                    out_v[d, pl.ds(g * LANES, LANES)] = vals * BETA_C + pd
                return dcarry

            lax.fori_loop(0, EMBED, d_body, 0)
            pltpu.sync_copy(out_v, out_hbm.at[s_pos, :, pl.ds(b0, CHUNK)])
            return carry

        lax.fori_loop(0, per_w, chunk_body, 0)

    return sc_kernel


def kernel(input_ids, table, pos_embedding):
    ids2d = input_ids.reshape(NCHUNKS, CHUNK).astype(jnp.int32)
    tab2 = table.reshape(table.shape[0] // 2, 2 * EMBED)
    out_t = _build_sc_kernel()(ids2d, tab2, pos_embedding)
    return jnp.transpose(out_t, (0, 2, 1))
